# Initial kernel scaffold; baseline (speedup 1.0000x reference)
import jax
import jax.numpy as jnp
from jax.experimental import pallas as pl


def kernel(x, S0, W_k, W_v, W_q, W_erase, W_write):
    raise NotImplementedError("write your pallas kernel here")


# speedup vs baseline: 3.2592x; 3.2592x over previous
import jax
import jax.numpy as jnp
from jax.experimental import pallas as pl
from jax.experimental.pallas import tpu as pltpu

EPS = 1e-6
BB = 16
TB = 64


def _ntm_block_kernel(x_ref, w_ref, s0_ref, out_ref, sfin_ref, s_ref, p_ref):
    n = s0_ref.shape[-1]
    tb = pl.program_id(1)

    @pl.when(tb == 0)
    def _():
        s_ref[...] = s0_ref[...]

    xb = x_ref[...].reshape(TB * BB, x_ref.shape[-1])
    p_ref[...] = jnp.dot(
        xb, w_ref[...], preferred_element_type=jnp.float32
    ).reshape(TB, BB, 5 * n)

    def step(t, carry):
        p = p_ref[t]
        k = p[:, 0:n]
        v = p[:, n:2 * n]
        q = p[:, 2 * n:3 * n]
        e = p[:, 3 * n:4 * n]
        w = p[:, 4 * n:5 * n]
        knorm = k / (jnp.sqrt(jnp.sum(k * k, axis=-1, keepdims=True)) + EPS)
        s = s_ref[...]
        eo = e[:, :, None] * knorm[:, None, :]
        wo = (w * v)[:, :, None] * knorm[:, None, :]
        s_new = jnp.tanh(s * (1.0 - eo) + wo)
        s_ref[...] = s_new
        sq = jnp.sum(s_new * q[:, None, :], axis=-1)
        out_ref[t] = sq * sq * jax.nn.sigmoid(sq)
        return carry

    jax.lax.fori_loop(0, TB, step, 0)

    @pl.when(tb == pl.num_programs(1) - 1)
    def _():
        sfin_ref[...] = s_ref[...]


def kernel(x, S0, W_k, W_v, W_q, W_erase, W_write):
    T, B, D = x.shape
    n = W_k.shape[0]
    wall = jnp.concatenate([W_k, W_v, W_q, W_erase, W_write], axis=0).T

    outputs, s_final = pl.pallas_call(
        _ntm_block_kernel,
        grid=(B // BB, T // TB),
        in_specs=[
            pl.BlockSpec((TB, BB, D), lambda b, t: (t, b, 0)),
            pl.BlockSpec((D, 5 * n), lambda b, t: (0, 0)),
            pl.BlockSpec((BB, n, n), lambda b, t: (b, 0, 0)),
        ],
        out_specs=[
            pl.BlockSpec((TB, BB, n), lambda b, t: (t, b, 0)),
            pl.BlockSpec((BB, n, n), lambda b, t: (b, 0, 0)),
        ],
        out_shape=[
            jax.ShapeDtypeStruct((T, B, n), jnp.float32),
            jax.ShapeDtypeStruct((B, n, n), jnp.float32),
        ],
        scratch_shapes=[
            pltpu.VMEM((BB, n, n), jnp.float32),
            pltpu.VMEM((TB, BB, 5 * n), jnp.float32),
        ],
        compiler_params=pltpu.CompilerParams(
            dimension_semantics=("parallel", "arbitrary"),
            vmem_limit_bytes=56 * 1024 * 1024,
        ),
        name="ntm_cell",
    )(x, wall, S0)
    return outputs, s_final


# speedup vs baseline: 4.7442x; 1.4557x over previous
import jax
import jax.numpy as jnp
from jax.experimental import pallas as pl
from jax.experimental.pallas import tpu as pltpu

EPS = 1e-6
BB = 16
TB = 64


def _ntm_block_kernel(x_ref, w_ref, s0_ref, out_ref, sfin_ref, s_ref, p_ref):
    n = s0_ref.shape[-1]
    tb = pl.program_id(1)

    @pl.when(tb == 0)
    def _():
        s_ref[...] = s0_ref[...]

    xb = x_ref[...].reshape(TB * BB, x_ref.shape[-1])
    p_ref[...] = jnp.dot(
        xb, w_ref[...], preferred_element_type=jnp.float32
    ).reshape(TB, BB, 5 * n)

    def step(t, carry):
        p = p_ref[t]
        k = p[:, 0:n]
        v = p[:, n:2 * n]
        q = p[:, 2 * n:3 * n]
        e = p[:, 3 * n:4 * n]
        w = p[:, 4 * n:5 * n]
        knorm = k / (jnp.sqrt(jnp.sum(k * k, axis=-1, keepdims=True)) + EPS)
        kb = knorm[:, :, None]
        e3 = e[:, None, :]
        wv3 = (w * v)[:, None, :]
        st = s_ref[...]
        st_new = jnp.tanh(st - kb * (st * e3 - wv3))
        s_ref[...] = st_new
        sq = jax.lax.dot_general(
            q[:, None, :], st_new,
            dimension_numbers=(((2,), (1,)), ((0,), (0,))),
            preferred_element_type=jnp.float32,
        ).reshape(BB, n)
        out_ref[t] = sq * sq * jax.nn.sigmoid(sq)
        return carry

    jax.lax.fori_loop(0, TB, step, 0)

    @pl.when(tb == pl.num_programs(1) - 1)
    def _():
        sfin_ref[...] = s_ref[...]


def kernel(x, S0, W_k, W_v, W_q, W_erase, W_write):
    T, B, D = x.shape
    n = W_k.shape[0]
    wall = jnp.concatenate([W_k, W_v, W_q, W_erase, W_write], axis=0).T
    s0t = jnp.swapaxes(S0, 1, 2)

    outputs, s_final_t = pl.pallas_call(
        _ntm_block_kernel,
        grid=(B // BB, T // TB),
        in_specs=[
            pl.BlockSpec((TB, BB, D), lambda b, t: (t, b, 0)),
            pl.BlockSpec((D, 5 * n), lambda b, t: (0, 0)),
            pl.BlockSpec((BB, n, n), lambda b, t: (b, 0, 0)),
        ],
        out_specs=[
            pl.BlockSpec((TB, BB, n), lambda b, t: (t, b, 0)),
            pl.BlockSpec((BB, n, n), lambda b, t: (b, 0, 0)),
        ],
        out_shape=[
            jax.ShapeDtypeStruct((T, B, n), jnp.float32),
            jax.ShapeDtypeStruct((B, n, n), jnp.float32),
        ],
        scratch_shapes=[
            pltpu.VMEM((BB, n, n), jnp.float32),
            pltpu.VMEM((TB, BB, 5 * n), jnp.float32),
        ],
        compiler_params=pltpu.CompilerParams(
            dimension_semantics=("parallel", "arbitrary"),
            vmem_limit_bytes=56 * 1024 * 1024,
        ),
        name="ntm_cell",
    )(x, wall, s0t)
    return outputs, jnp.swapaxes(s_final_t, 1, 2)


# speedup vs baseline: 6.3745x; 1.3436x over previous
import jax
import jax.numpy as jnp
from jax.experimental import pallas as pl
from jax.experimental.pallas import tpu as pltpu

EPS = 1e-6
BB = 16
TB = 64


def _ntm_block_kernel(x_ref, w_ref, s0_ref, out_ref, sfin_ref, s_ref, p_ref):
    n = s0_ref.shape[-1]
    tb = pl.program_id(1)

    @pl.when(tb == 0)
    def _():
        s_ref[...] = s0_ref[...]

    xb = x_ref[...].reshape(TB * BB, x_ref.shape[-1])
    p_ref[...] = jnp.dot(
        xb, w_ref[...], preferred_element_type=jnp.float32
    ).reshape(TB, BB, 5 * n)

    def math_step(st, t):
        p = p_ref[t]
        k = p[:, 0:n]
        v = p[:, n:2 * n]
        q = p[:, 2 * n:3 * n]
        e = p[:, 3 * n:4 * n]
        w = p[:, 4 * n:5 * n]
        knorm = k / (jnp.sqrt(jnp.sum(k * k, axis=-1, keepdims=True)) + EPS)
        kb = knorm[:, :, None]
        e3 = e[:, None, :]
        wv3 = (w * v)[:, None, :]
        st_new = jnp.tanh(st - kb * (st * e3 - wv3))
        sq = jax.lax.dot_general(
            q[:, None, :], st_new,
            dimension_numbers=(((2,), (1,)), ((0,), (0,))),
            preferred_element_type=jnp.float32,
        ).reshape(BB, n)
        return st_new, sq

    def step2(i, carry):
        t0 = 2 * i
        st0 = s_ref[...]
        st1, sq1 = math_step(st0, t0)
        st2, sq2 = math_step(st1, t0 + 1)
        s_ref[...] = st2
        out_ref[t0] = sq1 * sq1 * jax.nn.sigmoid(sq1)
        out_ref[t0 + 1] = sq2 * sq2 * jax.nn.sigmoid(sq2)
        return carry

    jax.lax.fori_loop(0, TB // 2, step2, 0)

    @pl.when(tb == pl.num_programs(1) - 1)
    def _():
        sfin_ref[...] = s_ref[...]


def kernel(x, S0, W_k, W_v, W_q, W_erase, W_write):
    T, B, D = x.shape
    n = W_k.shape[0]
    wall = jnp.concatenate([W_k, W_v, W_q, W_erase, W_write], axis=0).T
    s0t = jnp.swapaxes(S0, 1, 2)

    outputs, s_final_t = pl.pallas_call(
        _ntm_block_kernel,
        grid=(B // BB, T // TB),
        in_specs=[
            pl.BlockSpec((TB, BB, D), lambda b, t: (t, b, 0)),
            pl.BlockSpec((D, 5 * n), lambda b, t: (0, 0)),
            pl.BlockSpec((BB, n, n), lambda b, t: (b, 0, 0)),
        ],
        out_specs=[
            pl.BlockSpec((TB, BB, n), lambda b, t: (t, b, 0)),
            pl.BlockSpec((BB, n, n), lambda b, t: (b, 0, 0)),
        ],
        out_shape=[
            jax.ShapeDtypeStruct((T, B, n), jnp.float32),
            jax.ShapeDtypeStruct((B, n, n), jnp.float32),
        ],
        scratch_shapes=[
            pltpu.VMEM((BB, n, n), jnp.float32),
            pltpu.VMEM((TB, BB, 5 * n), jnp.float32),
        ],
        compiler_params=pltpu.CompilerParams(
            dimension_semantics=("parallel", "arbitrary"),
            vmem_limit_bytes=56 * 1024 * 1024,
        ),
        name="ntm_cell",
    )(x, wall, s0t)
    return outputs, jnp.swapaxes(s_final_t, 1, 2)


# speedup vs baseline: 6.8405x; 1.0731x over previous
import jax
import jax.numpy as jnp
from jax.experimental import pallas as pl
from jax.experimental.pallas import tpu as pltpu

EPS = 1e-6
BB = 16
TB = 64


def _ntm_block_kernel(x_ref, w_ref, s0_ref, out_ref, sfin_ref, s_ref, p_ref):
    n = s0_ref.shape[-1]
    tb = pl.program_id(1)

    @pl.when(tb == 0)
    def _():
        s_ref[...] = s0_ref[...]

    xb = x_ref[...].reshape(TB * BB, x_ref.shape[-1])
    p_ref[...] = jnp.dot(
        xb, w_ref[...], preferred_element_type=jnp.float32
    ).reshape(TB, BB, 5 * n)

    def math_step(st, t):
        p = p_ref[t]
        k = p[:, 0:n]
        v = p[:, n:2 * n]
        q = p[:, 2 * n:3 * n]
        e = p[:, 3 * n:4 * n]
        w = p[:, 4 * n:5 * n]
        knorm = k / (jnp.sqrt(jnp.sum(k * k, axis=-1, keepdims=True)) + EPS)
        kb = knorm[:, :, None]
        e3 = e[:, None, :]
        wv3 = (w * v)[:, None, :]
        st_new = jnp.tanh(st - kb * (st * e3 - wv3))
        sq = jax.lax.dot_general(
            q[:, None, :], st_new,
            dimension_numbers=(((2,), (1,)), ((0,), (0,))),
            preferred_element_type=jnp.float32,
        ).reshape(BB, n)
        return st_new, sq

    def step4(i, carry):
        t0 = 4 * i
        st = s_ref[...]
        for u in range(4):
            st, sq = math_step(st, t0 + u)
            out_ref[t0 + u] = sq * sq * jax.nn.sigmoid(sq)
        s_ref[...] = st
        return carry

    jax.lax.fori_loop(0, TB // 4, step4, 0)

    @pl.when(tb == pl.num_programs(1) - 1)
    def _():
        sfin_ref[...] = s_ref[...]


def kernel(x, S0, W_k, W_v, W_q, W_erase, W_write):
    T, B, D = x.shape
    n = W_k.shape[0]
    wall = jnp.concatenate([W_k, W_v, W_q, W_erase, W_write], axis=0).T
    s0t = jnp.swapaxes(S0, 1, 2)

    outputs, s_final_t = pl.pallas_call(
        _ntm_block_kernel,
        grid=(B // BB, T // TB),
        in_specs=[
            pl.BlockSpec((TB, BB, D), lambda b, t: (t, b, 0)),
            pl.BlockSpec((D, 5 * n), lambda b, t: (0, 0)),
            pl.BlockSpec((BB, n, n), lambda b, t: (b, 0, 0)),
        ],
        out_specs=[
            pl.BlockSpec((TB, BB, n), lambda b, t: (t, b, 0)),
            pl.BlockSpec((BB, n, n), lambda b, t: (b, 0, 0)),
        ],
        out_shape=[
            jax.ShapeDtypeStruct((T, B, n), jnp.float32),
            jax.ShapeDtypeStruct((B, n, n), jnp.float32),
        ],
        scratch_shapes=[
            pltpu.VMEM((BB, n, n), jnp.float32),
            pltpu.VMEM((TB, BB, 5 * n), jnp.float32),
        ],
        compiler_params=pltpu.CompilerParams(
            dimension_semantics=("parallel", "arbitrary"),
            vmem_limit_bytes=56 * 1024 * 1024,
        ),
        name="ntm_cell",
    )(x, wall, s0t)
    return outputs, jnp.swapaxes(s_final_t, 1, 2)


# speedup vs baseline: 7.1059x; 1.0388x over previous
import jax
import jax.numpy as jnp
from jax.experimental import pallas as pl
from jax.experimental.pallas import tpu as pltpu

EPS = 1e-6
BB = 16
TB = 64


def _ntm_block_kernel(x_ref, w_ref, s0_ref, out_ref, sfin_ref, s_ref, p_ref):
    n = s0_ref.shape[-1]
    tb = pl.program_id(1)

    @pl.when(tb == 0)
    def _():
        s_ref[...] = s0_ref[...]

    xb = x_ref[...].reshape(TB * BB, x_ref.shape[-1])
    p_ref[...] = jnp.dot(
        xb, w_ref[...], preferred_element_type=jnp.float32
    ).reshape(TB, BB, 5 * n)

    def math_step(st, t):
        p = p_ref[t]
        k = p[:, 0:n]
        v = p[:, n:2 * n]
        q = p[:, 2 * n:3 * n]
        e = p[:, 3 * n:4 * n]
        w = p[:, 4 * n:5 * n]
        knorm = k / (jnp.sqrt(jnp.sum(k * k, axis=-1, keepdims=True)) + EPS)
        kb = knorm[:, :, None]
        e3 = e[:, None, :]
        wv3 = (w * v)[:, None, :]
        st_new = jnp.tanh(st - kb * (st * e3 - wv3))
        sq = jax.lax.dot_general(
            q[:, None, :], st_new,
            dimension_numbers=(((2,), (1,)), ((0,), (0,))),
            preferred_element_type=jnp.float32,
        ).reshape(BB, n)
        return st_new, sq

    def step4(i, carry):
        t0 = 8 * i
        st = s_ref[...]
        for u in range(8):
            st, sq = math_step(st, t0 + u)
            out_ref[t0 + u] = sq * sq * jax.nn.sigmoid(sq)
        s_ref[...] = st
        return carry

    jax.lax.fori_loop(0, TB // 8, step4, 0)

    @pl.when(tb == pl.num_programs(1) - 1)
    def _():
        sfin_ref[...] = s_ref[...]


def kernel(x, S0, W_k, W_v, W_q, W_erase, W_write):
    T, B, D = x.shape
    n = W_k.shape[0]
    wall = jnp.concatenate([W_k, W_v, W_q, W_erase, W_write], axis=0).T
    s0t = jnp.swapaxes(S0, 1, 2)

    outputs, s_final_t = pl.pallas_call(
        _ntm_block_kernel,
        grid=(B // BB, T // TB),
        in_specs=[
            pl.BlockSpec((TB, BB, D), lambda b, t: (t, b, 0)),
            pl.BlockSpec((D, 5 * n), lambda b, t: (0, 0)),
            pl.BlockSpec((BB, n, n), lambda b, t: (b, 0, 0)),
        ],
        out_specs=[
            pl.BlockSpec((TB, BB, n), lambda b, t: (t, b, 0)),
            pl.BlockSpec((BB, n, n), lambda b, t: (b, 0, 0)),
        ],
        out_shape=[
            jax.ShapeDtypeStruct((T, B, n), jnp.float32),
            jax.ShapeDtypeStruct((B, n, n), jnp.float32),
        ],
        scratch_shapes=[
            pltpu.VMEM((BB, n, n), jnp.float32),
            pltpu.VMEM((TB, BB, 5 * n), jnp.float32),
        ],
        compiler_params=pltpu.CompilerParams(
            dimension_semantics=("parallel", "arbitrary"),
            vmem_limit_bytes=56 * 1024 * 1024,
        ),
        name="ntm_cell",
    )(x, wall, s0t)
    return outputs, jnp.swapaxes(s_final_t, 1, 2)
